# Initial kernel scaffold; baseline (speedup 1.0000x reference)
import jax
import jax.numpy as jnp
from jax.experimental import pallas as pl


def kernel(z, edge_index, bond_type, x_clique, node2clique_index, clique_edge_index, params):
    raise NotImplementedError("write your pallas kernel here")


# speedup vs baseline: 1.2246x; 1.2246x over previous
import functools

import jax
import jax.numpy as jnp
from jax import lax
from jax.experimental import pallas as pl

_N = 10000
_E = 160000
_NC = 4000
_EN2C = 10000
_ECC = 8000
_H = 64
_M = 4
_CH = 32
_MAXB = 8
_L = 2
_AVG_LOG = 2.833213344056216


def _tc(fn, out_shape, *args):
    return pl.pallas_call(fn, out_shape=out_shape)(*args)


def _prologue_body(ng_ref, cg_ref, nW_ref, nb_ref, cW_ref, cb_ref, bt_ref,
                   we1s_ref, be1s_ref, we2s_ref, be2s_ref, wc2s_ref, bc1s_ref, bc2s_ref,
                   x0_ref, c0_ref, wbm_ref, wce_ref):
    x0_ref[...] = jax.nn.relu(
        jnp.dot(jax.nn.relu(ng_ref[...]), nW_ref[...],
                preferred_element_type=jnp.float32) + nb_ref[...])
    c0_ref[...] = jax.nn.relu(
        jnp.dot(jax.nn.relu(cg_ref[...]), cW_ref[...],
                preferred_element_type=jnp.float32) + cb_ref[...])
    for l in range(_L):
        hb = jax.nn.relu(jnp.dot(bt_ref[...], we1s_ref[l],
                                 preferred_element_type=jnp.float32) + be1s_ref[l])
        wb = jnp.dot(hb, we2s_ref[l],
                     preferred_element_type=jnp.float32) + be2s_ref[l]
        wbm_ref[l, ...] = wb.reshape(_MAXB, _H, _M).transpose(1, 0, 2).reshape(_H, _MAXB * _M)
        chh = jax.nn.relu(bc1s_ref[l])
        wce_ref[l, ...] = (jnp.dot(chh, wc2s_ref[l],
                                   preferred_element_type=jnp.float32)
                           + bc2s_ref[l]).reshape(_CH, _CH)


def _pna_post_body(x_ref, ssum_ref, ssq_ref, smax_ref, smin_ref, deg_ref,
                   wp_ref, bp_ref, out_ref):
    deg = deg_ref[...]
    degc = jnp.maximum(deg, 1.0)
    mean = ssum_ref[...] / degc
    has = deg > 0.0
    mx = jnp.where(has, smax_ref[...], 0.0)
    mn = jnp.where(has, smin_ref[...], 0.0)
    std = jnp.sqrt(jnp.clip(ssq_ref[...] / degc - mean * mean, 0.0, None))
    agg = jnp.concatenate([mean, mx, mn, std], axis=-1)
    logd = jnp.log(deg + 1.0)
    amp = logd / _AVG_LOG
    att = jnp.where(logd > 0, _AVG_LOG / jnp.clip(logd, 1e-6, None), 0.0)
    scaled = jnp.concatenate([agg, agg * amp, agg * att], axis=-1)
    h = jnp.concatenate([x_ref[...], scaled], axis=-1)
    out_ref[...] = jax.nn.relu(
        jnp.dot(h, wp_ref[...], preferred_element_type=jnp.float32) + bp_ref[...])


def _c_update_body(c_ref, caggs_ref, cdeg_ref, wn2c_ref, bn2c_ref, wce_ref,
                   c1_ref, cy_ref):
    cagg = caggs_ref[...] / cdeg_ref[...]
    c1 = c_ref[...] + jax.nn.relu(
        jnp.dot(cagg, wn2c_ref[...], preferred_element_type=jnp.float32) + bn2c_ref[...])
    c1_ref[...] = c1
    cy_ref[...] = jnp.dot(c1, wce_ref[...], preferred_element_type=jnp.float32)


def _c_root_body(c_ref, cas_ref, ccdeg_ref, wroot_ref, broot_ref, out_ref):
    out_ref[...] = (jnp.dot(c_ref[...], wroot_ref[...],
                            preferred_element_type=jnp.float32)
                    + cas_ref[...] / ccdeg_ref[...] + broot_ref[...])


def _x_update_body(x_ref, naggs_ref, ndeg_ref, wc2n_ref, bc2n_ref, out_ref):
    nagg = naggs_ref[...] / ndeg_ref[...]
    out_ref[...] = x_ref[...] + jax.nn.relu(
        jnp.dot(nagg, wc2n_ref[...], preferred_element_type=jnp.float32) + bc2n_ref[...])


def _f32(shape):
    return jax.ShapeDtypeStruct(shape, jnp.float32)


def kernel(z, edge_index, bond_type, x_clique, node2clique_index, clique_edge_index, params):
    emb = params["emb"]
    layers = params["layers"]

    src, dst = edge_index[0], edge_index[1]
    csrc, cdst = clique_edge_index[0], clique_edge_index[1]
    nn2c, cn2c = node2clique_index[0], node2clique_index[1]

    ng = jnp.take(emb["node_table"], z, axis=0)
    cg = jnp.take(emb["clique_table"], x_clique, axis=0)

    onesE = jnp.ones((_E,), jnp.float32)
    ones_n2c = jnp.ones((_EN2C,), jnp.float32)
    deg = jax.ops.segment_sum(onesE, dst, num_segments=_N)[:, None]
    cdeg = jnp.maximum(jax.ops.segment_sum(ones_n2c, cn2c, num_segments=_NC), 1.0)[:, None]
    ccdeg = jnp.maximum(jax.ops.segment_sum(jnp.ones((_ECC,), jnp.float32), cdst,
                                            num_segments=_NC), 1.0)[:, None]
    ndeg = jnp.maximum(jax.ops.segment_sum(ones_n2c, nn2c, num_segments=_N), 1.0)[:, None]

    gidx = src * _MAXB + bond_type

    we1s = jnp.stack([p["We1"] for p in layers])
    be1s = jnp.stack([p["be1"][None, :] for p in layers])
    we2s = jnp.stack([p["We2"] for p in layers])
    be2s = jnp.stack([p["be2"][None, :] for p in layers])
    wc2s = jnp.stack([p["Wc2"] for p in layers])
    bc1s = jnp.stack([p["bc1"][None, :] for p in layers])
    bc2s = jnp.stack([p["bc2"][None, :] for p in layers])

    x0, c0, wbm, wce = _tc(
        _prologue_body,
        (_f32((_N, _H)), _f32((_NC, _CH)),
         _f32((_L, _H, _MAXB * _M)), _f32((_L, _CH, _CH))),
        ng, cg, emb["node_linW"], emb["node_linb"][None, :],
        emb["clique_linW"], emb["clique_linb"][None, :], emb["bond_table"],
        we1s, be1s, we2s, be2s, wc2s, bc1s, bc2s)

    x, c = x0, c0
    for l in range(_L):
        p = layers[l]
        y = x @ wbm[l]
        msgs = y.reshape(_N * _MAXB, _M)[gidx]
        ssum = jax.ops.segment_sum(msgs, dst, num_segments=_N)
        ssq = jax.ops.segment_sum(msgs * msgs, dst, num_segments=_N)
        smax = jax.ops.segment_max(msgs, dst, num_segments=_N)
        smin = jax.ops.segment_min(msgs, dst, num_segments=_N)
        nb = 2000
        row = lambda w: pl.BlockSpec((nb, w), lambda i: (i, 0))
        full = lambda a, b: pl.BlockSpec((a, b), lambda i: (0, 0))
        x = pl.pallas_call(
            _pna_post_body,
            grid=(_N // nb,),
            in_specs=[row(_H), row(_M), row(_M), row(_M), row(_M), row(1),
                      full(_H + 12 * _M, _H), full(1, _H)],
            out_specs=row(_H),
            out_shape=_f32((_N, _H)),
        )(x, ssum, ssq, smax, smin, deg, p["Wp"], p["bp"][None, :])

        caggs = jax.ops.segment_sum(x[nn2c], cn2c, num_segments=_NC)
        c, cy = _tc(_c_update_body, (_f32((_NC, _CH)), _f32((_NC, _CH))),
                    c, caggs, cdeg, p["Wn2c"], p["bn2c"][None, :], wce[l])

        cas = jax.ops.segment_sum(cy[csrc], cdst, num_segments=_NC)
        c = _tc(_c_root_body, _f32((_NC, _CH)),
                c, cas, ccdeg, p["Wroot"], p["broot"][None, :])

        naggs = jax.ops.segment_sum(c[cn2c], nn2c, num_segments=_N)
        x = _tc(_x_update_body, _f32((_N, _H)),
                x, naggs, ndeg, p["Wc2n"], p["bc2n"][None, :])

    return x, c


# speedup vs baseline: 4.1025x; 3.3501x over previous
import functools

import jax
import jax.numpy as jnp
from jax import lax
from jax.experimental import pallas as pl
from jax.experimental.pallas import tpu as pltpu
from jax.experimental.pallas import tpu_sc as plsc

_N = 10000
_E = 160000
_NC = 4000
_EN2C = 10000
_ECC = 8000
_H = 64
_M = 4
_CH = 32
_MAXB = 8
_L = 2
_AVG_LOG = 2.833213344056216

_NW = 32
_NSEGN = 10112
_NSEGC = 4096
_EPT = _E // _NW
_ECH = 1000

_mesh = plsc.VectorSubcoreMesh(core_axis_name="c", subcore_axis_name="s")
_sc_params = pltpu.CompilerParams(needs_layout_passes=False,
                                  use_tc_tiling_on_sc=False)


def _f32(shape):
    return jax.ShapeDtypeStruct(shape, jnp.float32)


_SUMW = _NSEGN * 12
_S16 = _SUMW // 16
_NP4 = 40960
_SL = _NP4 // 16


def _sc_edge(y1, gidx4, d4, ia, ib, ic, neg, pos, ones1, zsum):
    kern = pl.kernel(
        _sc_edge_body,
        out_type=(_f32((2 * _SUMW,)), _f32((2 * _NP4,)), _f32((2 * _NP4,))),
        mesh=_mesh,
        scratch_types=[
            pltpu.VMEM((_ECH * 4,), jnp.int32),
            pltpu.VMEM((_ECH * 4,), jnp.int32),
            pltpu.VMEM((_ECH * 4,), jnp.int32),
            pltpu.VMEM((_ECH * 4,), jnp.int32),
            pltpu.VMEM((_ECH,), jnp.int32),
            pltpu.VMEM((_ECH * 4,), jnp.float32),
            pltpu.VMEM((_ECH * 4,), jnp.float32),
            pltpu.VMEM((_ECH,), jnp.float32),
            pltpu.VMEM((_NP4,), jnp.float32),
            pltpu.VMEM((_NP4,), jnp.float32),
            pltpu.VMEM((_S16,), jnp.float32),
            pltpu.VMEM_SHARED((_SUMW,), jnp.float32),
            pltpu.VMEM_SHARED((16 * _SL,), jnp.float32),
            pltpu.SemaphoreType.DMA,
        ],
        compiler_params=_sc_params,
    )
    return kern(y1, gidx4, d4, ia, ib, ic, neg, pos, ones1, zsum)


def _sc_edge_body(y_h, g_h, d_h, ia_h, ib_h, ic_h, neg_h, pos_h, on_h, z_h,
                  sums_h, mx_h, mn_h,
                  gv, dv, iav, ibv, icv, rows, sq, ov, amax, amin, zb,
                  acc, shx, sem):
    c = lax.axis_index("c")
    s = lax.axis_index("s")
    wid = s * 2 + c

    pltpu.sync_copy(neg_h, amax)
    pltpu.sync_copy(pos_h, amin)
    pltpu.sync_copy(on_h, ov)
    pltpu.sync_copy(z_h, zb)
    pltpu.sync_copy(zb, acc.at[pl.ds(s * _S16, _S16)])
    plsc.subcore_barrier()

    for ch in range(_EPT // _ECH):
        base = wid * _EPT + ch * _ECH
        pltpu.sync_copy(g_h.at[pl.ds(base * 4, _ECH * 4)], gv)
        pltpu.sync_copy(d_h.at[pl.ds(base * 4, _ECH * 4)], dv)
        pltpu.sync_copy(ia_h.at[pl.ds(base * 4, _ECH * 4)], iav)
        pltpu.sync_copy(ib_h.at[pl.ds(base * 4, _ECH * 4)], ibv)
        pltpu.sync_copy(ic_h.at[pl.ds(base, _ECH)], icv)
        pltpu.async_copy(y_h.at[gv], rows, sem).wait()

        def body(i, carry):
            m = rows[pl.ds(i * 16, 16)]
            d4 = dv[pl.ds(i * 16, 16)]
            sq[pl.ds(i * 16, 16)] = m * m
            cur = plsc.load_gather(amax, [d4])
            new = jnp.maximum(cur, m)
            plsc.store_scatter(amax, [d4], new)
            curn = plsc.load_gather(amin, [d4])
            newn = jnp.minimum(curn, m)
            plsc.store_scatter(amin, [d4], newn)
            chk = plsc.load_gather(amax, [d4])
            chkn = plsc.load_gather(amin, [d4])
            bad = jnp.any((chk < new) | (chkn > newn))

            @pl.when(bad)
            def _():
                for _ in range(3):
                    c2 = plsc.load_gather(amax, [d4])
                    plsc.store_scatter(amax, [d4], jnp.maximum(c2, m))
                    c3 = plsc.load_gather(amin, [d4])
                    plsc.store_scatter(amin, [d4], jnp.minimum(c3, m))
            return carry

        lax.fori_loop(0, _ECH * 4 // 16, body, 0)
        pltpu.sync_copy(rows, acc.at[iav], add=True)
        pltpu.sync_copy(sq, acc.at[ibv], add=True)
        pltpu.sync_copy(ov, acc.at[icv], add=True)

    pltpu.sync_copy(neg_h.at[pl.ds(0, _SL)], rows.at[pl.ds(0, _SL)])
    for r in range(16):
        j = (s + r) & 15
        pltpu.sync_copy(amax.at[pl.ds(j * _SL, _SL)],
                        shx.at[pl.ds(j * _SL, _SL)])
        plsc.subcore_barrier()
        pltpu.sync_copy(shx.at[pl.ds(s * _SL, _SL)], zb.at[pl.ds(0, _SL)])

        def mbody(k, carry):
            rows[pl.ds(k * 16, 16)] = jnp.maximum(rows[pl.ds(k * 16, 16)],
                                                  zb[pl.ds(k * 16, 16)])
            return carry
        lax.fori_loop(0, _SL // 16, mbody, 0)
        plsc.subcore_barrier()
    pltpu.sync_copy(rows.at[pl.ds(0, _SL)],
                    mx_h.at[pl.ds(c * _NP4 + s * _SL, _SL)])

    pltpu.sync_copy(pos_h.at[pl.ds(0, _SL)], sq.at[pl.ds(0, _SL)])
    for r in range(16):
        j = (s + r) & 15
        pltpu.sync_copy(amin.at[pl.ds(j * _SL, _SL)],
                        shx.at[pl.ds(j * _SL, _SL)])
        plsc.subcore_barrier()
        pltpu.sync_copy(shx.at[pl.ds(s * _SL, _SL)], zb.at[pl.ds(0, _SL)])

        def nbody(k, carry):
            sq[pl.ds(k * 16, 16)] = jnp.minimum(sq[pl.ds(k * 16, 16)],
                                                zb[pl.ds(k * 16, 16)])
            return carry
        lax.fori_loop(0, _SL // 16, nbody, 0)
        plsc.subcore_barrier()
    pltpu.sync_copy(sq.at[pl.ds(0, _SL)],
                    mn_h.at[pl.ds(c * _NP4 + s * _SL, _SL)])

    pltpu.sync_copy(acc.at[pl.ds(s * _S16, _S16)], zb)
    pltpu.sync_copy(zb, sums_h.at[pl.ds(c * _SUMW + s * _S16, _S16)])


def _sc_gather_segsum(table, sidx, didx, onesv, zd, zc, nseg, d,
                      with_counts=True):
    eq = sidx.shape[0]
    per = eq // _NW
    nseg16 = nseg // 16

    def body(t_h, s_h, di_h, on_h, zd_h, zc_h, *rest):
        if with_counts:
            (out_h, cnt_h, sv, dv, rows, ov, zbd, zbc, acc, accc, sem) = rest
        else:
            (out_h, sv, dv, rows, ov, zbd, zbc, acc, sem) = rest
        c = lax.axis_index("c")
        s = lax.axis_index("s")
        wid = s * 2 + c
        base = wid * per
        pltpu.sync_copy(zd_h, zbd)
        pltpu.sync_copy(zbd, acc.at[pl.ds(s * nseg16, nseg16)])
        if with_counts:
            pltpu.sync_copy(zc_h, zbc)
            pltpu.sync_copy(zbc, accc.at[pl.ds(s * nseg16, nseg16)])
            pltpu.sync_copy(on_h, ov)
        plsc.subcore_barrier()
        pltpu.sync_copy(s_h.at[pl.ds(base, per)], sv)
        pltpu.sync_copy(di_h.at[pl.ds(base, per)], dv)
        pltpu.async_copy(t_h.at[sv], rows, sem).wait()
        pltpu.sync_copy(rows, acc.at[dv], add=True)
        if with_counts:
            pltpu.sync_copy(ov, accc.at[dv], add=True)
        plsc.subcore_barrier()
        pltpu.sync_copy(acc.at[pl.ds(s * nseg16, nseg16)], zbd)
        pltpu.sync_copy(zbd, out_h.at[c, pl.ds(s * nseg16, nseg16)])
        if with_counts:
            pltpu.sync_copy(accc.at[pl.ds(s * nseg16, nseg16)], zbc)
            pltpu.sync_copy(zbc, cnt_h.at[c, pl.ds(s * nseg16, nseg16)])

    out_type = [_f32((2, nseg, d))]
    scratch = [
        pltpu.VMEM((per,), jnp.int32),
        pltpu.VMEM((per,), jnp.int32),
        pltpu.VMEM((per, d), jnp.float32),
        pltpu.VMEM((per, 8), jnp.float32),
        pltpu.VMEM((nseg16, d), jnp.float32),
        pltpu.VMEM((nseg16, 8), jnp.float32),
        pltpu.VMEM_SHARED((nseg, d), jnp.float32),
    ]
    if with_counts:
        out_type.append(_f32((2, nseg, 8)))
        scratch.append(pltpu.VMEM_SHARED((nseg, 8), jnp.float32))
    scratch.append(pltpu.SemaphoreType.DMA)

    kern = pl.kernel(
        body,
        out_type=tuple(out_type),
        mesh=_mesh,
        scratch_types=scratch,
        compiler_params=_sc_params,
    )
    res = kern(table, sidx, didx, onesv, zd, zc)
    if with_counts:
        return res
    return res[0], None


def _prologue_body(ng_ref, cg_ref, nW_ref, nb_ref, cW_ref, cb_ref, bt_ref,
                   we1s_ref, be1s_ref, we2s_ref, be2s_ref, wc2s_ref, bc1s_ref,
                   bc2s_ref, x0_ref, c0_ref, wbm_ref, wce_ref, y0_ref):
    x0 = jax.nn.relu(
        jnp.dot(jax.nn.relu(ng_ref[...]), nW_ref[...],
                preferred_element_type=jnp.float32) + nb_ref[...])
    x0_ref[...] = x0
    c0_ref[...] = jax.nn.relu(
        jnp.dot(jax.nn.relu(cg_ref[...]), cW_ref[...],
                preferred_element_type=jnp.float32) + cb_ref[...])
    for l in range(_L):
        hb = jax.nn.relu(jnp.dot(bt_ref[...], we1s_ref[l],
                                 preferred_element_type=jnp.float32) + be1s_ref[l])
        wb = jnp.dot(hb, we2s_ref[l],
                     preferred_element_type=jnp.float32) + be2s_ref[l]
        wbm_ref[l, ...] = wb.reshape(_MAXB, _H, _M).transpose(1, 0, 2).reshape(_H, _MAXB * _M)
        chh = jax.nn.relu(bc1s_ref[l])
        wce_ref[l, ...] = (jnp.dot(chh, wc2s_ref[l],
                                   preferred_element_type=jnp.float32)
                           + bc2s_ref[l]).reshape(_CH, _CH)
    y0_ref[...] = jnp.dot(x0, wbm_ref[0, ...], preferred_element_type=jnp.float32)


def _pna_post_body(x_ref, sums_ref, mxp_ref, mnp_ref, wp_ref, bp_ref, out_ref):
    ssum = sums_ref[0, :, 0:4] + sums_ref[1, :, 0:4]
    ssq = sums_ref[0, :, 4:8] + sums_ref[1, :, 4:8]
    deg = (sums_ref[0, :, 8] + sums_ref[1, :, 8])[:, None]
    degc = jnp.maximum(deg, 1.0)
    mean = ssum / degc
    has = deg > 0.0
    mx = jnp.where(has, jnp.max(mxp_ref[...], axis=0), 0.0)
    mn = jnp.where(has, jnp.min(mnp_ref[...], axis=0), 0.0)
    std = jnp.sqrt(jnp.clip(ssq / degc - mean * mean, 0.0, None))
    agg = jnp.concatenate([mean, mx, mn, std], axis=-1)
    logd = jnp.log(deg + 1.0)
    amp = logd / _AVG_LOG
    att = jnp.where(logd > 0, _AVG_LOG / jnp.clip(logd, 1e-6, None), 0.0)
    scaled = jnp.concatenate([agg, agg * amp, agg * att], axis=-1)
    h = jnp.concatenate([x_ref[...], scaled], axis=-1)
    out_ref[...] = jax.nn.relu(
        jnp.dot(h, wp_ref[...], preferred_element_type=jnp.float32) + bp_ref[...])


def _c_update_body(c_ref, caggs_ref, cnt_ref, wn2c_ref, bn2c_ref, wce_ref,
                   c1_ref, cy_ref):
    cdeg = jnp.maximum(cnt_ref[0, :, 0] + cnt_ref[1, :, 0], 1.0)[:, None]
    cagg = (caggs_ref[0] + caggs_ref[1]) / cdeg
    c1 = c_ref[...] + jax.nn.relu(
        jnp.dot(cagg, wn2c_ref[...], preferred_element_type=jnp.float32) + bn2c_ref[...])
    c1_ref[...] = c1
    cy_ref[...] = jnp.dot(c1, wce_ref[...], preferred_element_type=jnp.float32)


def _c_root_body(c_ref, cas_ref, cnt_ref, wroot_ref, broot_ref, out_ref):
    ccdeg = jnp.maximum(cnt_ref[0, :, 0] + cnt_ref[1, :, 0], 1.0)[:, None]
    out_ref[...] = (jnp.dot(c_ref[...], wroot_ref[...],
                            preferred_element_type=jnp.float32)
                    + (cas_ref[0] + cas_ref[1]) / ccdeg + broot_ref[...])


def _x_update_body(x_ref, naggs_ref, cnt_ref, wc2n_ref, bc2n_ref, wbm_ref,
                   out_ref, y_ref):
    ndeg = jnp.maximum(cnt_ref[0, :, 0] + cnt_ref[1, :, 0], 1.0)[:, None]
    nagg = (naggs_ref[0] + naggs_ref[1]) / ndeg
    xn = x_ref[...] + jax.nn.relu(
        jnp.dot(nagg, wc2n_ref[...], preferred_element_type=jnp.float32) + bc2n_ref[...])
    out_ref[...] = xn
    y_ref[...] = jnp.dot(xn, wbm_ref[...], preferred_element_type=jnp.float32)


def _tc(fn, out_shape, *args):
    return pl.pallas_call(fn, out_shape=out_shape)(*args)


def _pad_i32(a, n, fill):
    return jnp.concatenate([a, jnp.full((n - a.shape[0],), fill, jnp.int32)])


def kernel(z, edge_index, bond_type, x_clique, node2clique_index, clique_edge_index, params):
    emb = params["emb"]
    layers = params["layers"]

    src, dst = edge_index[0], edge_index[1]
    csrc, cdst = clique_edge_index[0], clique_edge_index[1]
    nn2c, cn2c = node2clique_index[0], node2clique_index[1]

    ng = jnp.take(emb["node_table"], z, axis=0)
    cg = jnp.take(emb["clique_table"], x_clique, axis=0)

    gidx = (src * _MAXB + bond_type).astype(jnp.int32)
    dst = dst.astype(jnp.int32)
    comp = jnp.arange(4, dtype=jnp.int32)[None, :]
    gidx4 = (gidx[:, None] * 4 + comp).reshape(-1)
    d4 = (dst[:, None] * 4 + comp).reshape(-1)
    ia = (dst[:, None] * 12 + comp).reshape(-1)
    ib = (dst[:, None] * 12 + 4 + comp).reshape(-1)
    ic = dst * 12 + 8

    en2c_p = 10240
    ecc_p = 8192
    nn2c_g = _pad_i32(nn2c.astype(jnp.int32), en2c_p, 0)
    cn2c_s = _pad_i32(cn2c.astype(jnp.int32), en2c_p, _NC)
    cn2c_g = _pad_i32(cn2c.astype(jnp.int32), en2c_p, 0)
    nn2c_s = _pad_i32(nn2c.astype(jnp.int32), en2c_p, _N)
    csrc_g = _pad_i32(csrc.astype(jnp.int32), ecc_p, 0)
    cdst_s = _pad_i32(cdst.astype(jnp.int32), ecc_p, _NC)

    neg = jnp.full((_NP4,), -3.4e38, jnp.float32)
    pos = jnp.full((_NP4,), 3.4e38, jnp.float32)
    ones1 = jnp.ones((_ECH,), jnp.float32)
    zsum = jnp.zeros((_S16,), jnp.float32)
    ones_n2c = jnp.ones((en2c_p // _NW, 8), jnp.float32)
    ones_ecc = jnp.ones((ecc_p // _NW, 8), jnp.float32)
    zn64 = jnp.zeros((_NSEGC // 16, _H), jnp.float32)
    zc32 = jnp.zeros((_NSEGC // 16, _CH), jnp.float32)
    zn32 = jnp.zeros((_NSEGN // 16, _CH), jnp.float32)
    zc8 = jnp.zeros((_NSEGC // 16, 8), jnp.float32)
    zn8 = jnp.zeros((_NSEGN // 16, 8), jnp.float32)

    we1s = jnp.stack([p["We1"] for p in layers])
    be1s = jnp.stack([p["be1"][None, :] for p in layers])
    we2s = jnp.stack([p["We2"] for p in layers])
    be2s = jnp.stack([p["be2"][None, :] for p in layers])
    wc2s = jnp.stack([p["Wc2"] for p in layers])
    bc1s = jnp.stack([p["bc1"][None, :] for p in layers])
    bc2s = jnp.stack([p["bc2"][None, :] for p in layers])

    x, c, wbm, wce, y = _tc(
        _prologue_body,
        (_f32((_N, _H)), _f32((_NC, _CH)),
         _f32((_L, _H, _MAXB * _M)), _f32((_L, _CH, _CH)), _f32((_N, _MAXB * _M))),
        ng, cg, emb["node_linW"], emb["node_linb"][None, :],
        emb["clique_linW"], emb["clique_linb"][None, :], emb["bond_table"],
        we1s, be1s, we2s, be2s, wc2s, bc1s, bc2s)

    for l in range(_L):
        p = layers[l]
        y1 = y.reshape(_N * _MAXB * _M)
        sums, mxp, mnp = _sc_edge(y1, gidx4, d4, ia, ib, ic, neg, pos,
                                  ones1, zsum)
        sums = sums.reshape(2, _NSEGN, 12)
        mxp = mxp.reshape(2, _NP4 // 4, 4)[:, :_N, :]
        mnp = mnp.reshape(2, _NP4 // 4, 4)[:, :_N, :]
        nb = 2000
        row = lambda w: pl.BlockSpec((nb, w), lambda i: (i, 0))
        x = pl.pallas_call(
            _pna_post_body,
            grid=(_N // nb,),
            in_specs=[row(_H),
                      pl.BlockSpec((2, nb, 12), lambda i: (0, i, 0)),
                      pl.BlockSpec((2, nb, 4), lambda i: (0, i, 0)),
                      pl.BlockSpec((2, nb, 4), lambda i: (0, i, 0)),
                      pl.BlockSpec((_H + 12 * _M, _H), lambda i: (0, 0)),
                      pl.BlockSpec((1, _H), lambda i: (0, 0))],
            out_specs=row(_H),
            out_shape=_f32((_N, _H)),
        )(x, sums[:, :_N, :], mxp, mnp, p["Wp"], p["bp"][None, :])

        caggs, ccnt_new = _sc_gather_segsum(x, nn2c_g, cn2c_s, ones_n2c,
                                            zn64, zc8, _NSEGC, _H,
                                            with_counts=(l == 0))
        if l == 0:
            ccnt = ccnt_new
        c, cy = _tc(_c_update_body, (_f32((_NC, _CH)), _f32((_NC, _CH))),
                    c, caggs[:, :_NC, :], ccnt[:, :_NC, :],
                    p["Wn2c"], p["bn2c"][None, :], wce[l])

        cas, cccnt_new = _sc_gather_segsum(cy, csrc_g, cdst_s, ones_ecc,
                                           zc32, zc8, _NSEGC, _CH,
                                           with_counts=(l == 0))
        if l == 0:
            cccnt = cccnt_new
        c = _tc(_c_root_body, _f32((_NC, _CH)),
                c, cas[:, :_NC, :], cccnt[:, :_NC, :],
                p["Wroot"], p["broot"][None, :])

        naggs, ncnt_new = _sc_gather_segsum(c, cn2c_g, nn2c_s, ones_n2c,
                                            zn32, zn8, _NSEGN, _CH,
                                            with_counts=(l == 0))
        if l == 0:
            ncnt = ncnt_new
        wb_next = wbm[l + 1] if l + 1 < _L else wbm[l]
        x, y = _tc(_x_update_body, (_f32((_N, _H)), _f32((_N, _MAXB * _M))),
                   x, naggs[:, :_N, :], ncnt[:, :_N, :],
                   p["Wc2n"], p["bc2n"][None, :], wb_next)

    return x, c


# speedup vs baseline: 8.2947x; 2.0219x over previous
import functools

import jax
import jax.numpy as jnp
from jax import lax
from jax.experimental import pallas as pl
from jax.experimental.pallas import tpu as pltpu
from jax.experimental.pallas import tpu_sc as plsc

_N = 10000
_E = 160000
_NC = 4000
_EN2C = 10000
_ECC = 8000
_H = 64
_M = 4
_CH = 32
_MAXB = 8
_L = 2
_AVG_LOG = 2.833213344056216

_NW = 32
_NSEGN = 10112
_NSEGC = 4096
_EPT = _E // _NW
_ECH = 1000

_mesh = plsc.VectorSubcoreMesh(core_axis_name="c", subcore_axis_name="s")
_sc_params = pltpu.CompilerParams(needs_layout_passes=False,
                                  use_tc_tiling_on_sc=False)


def _f32(shape):
    return jax.ShapeDtypeStruct(shape, jnp.float32)


_N16S = _NSEGN // 16
_NP8 = 81920
_SL = _NP8 // 16


def _sc_edge(y16, gidx, dst, cn2c_s, cdst_s, nn2c_s, c13, c14, c15,
             einit, z16):
    kern = pl.kernel(
        _sc_edge_body,
        out_type=(_f32((2, _NSEGN, 16)), _f32((_NW * _NP8,))),
        mesh=_mesh,
        scratch_types=[
            pltpu.VMEM((_ECH,), jnp.int32),
            pltpu.VMEM((_ECH,), jnp.int32),
            pltpu.VMEM((_ECH, 16), jnp.float32),
            pltpu.VMEM((_NP8,), jnp.float32),
            pltpu.VMEM((_N16S, 16), jnp.float32),
            pltpu.VMEM((320,), jnp.int32),
            pltpu.VMEM((320, 16), jnp.float32),
            pltpu.VMEM((256,), jnp.int32),
            pltpu.VMEM((256, 16), jnp.float32),
            pltpu.VMEM_SHARED((_NSEGN, 16), jnp.float32),
            pltpu.SemaphoreType.DMA,
        ],
        compiler_params=_sc_params,
    )
    return kern(y16, gidx, dst, cn2c_s, cdst_s, nn2c_s, c13, c14, c15,
                einit, z16)


def _sc_edge_body(y_h, g_h, d_h, cn_h, cd_h, nn_h, c13_h, c14_h, c15_h,
                  einit_h, z_h,
                  sums_h, ex_h,
                  gv, dv, rows, eacc, zb, div, dcb, div2, dcb2,
                  acc, sem):
    c = lax.axis_index("c")
    s = lax.axis_index("s")
    wid = s * 2 + c

    pltpu.sync_copy(einit_h, eacc)
    pltpu.sync_copy(z_h, zb)
    pltpu.sync_copy(zb, acc.at[pl.ds(s * _N16S, _N16S)])
    plsc.subcore_barrier()

    lane = lax.iota(jnp.int32, 16)
    lane7 = lane & 7
    msel = lane7 < 4
    m8 = lane < 8

    for ch in range(_EPT // _ECH):
        base = wid * _EPT + ch * _ECH
        pltpu.sync_copy(g_h.at[pl.ds(base, _ECH)], gv)
        pltpu.sync_copy(d_h.at[pl.ds(base, _ECH)], dv)
        pltpu.async_copy(y_h.at[gv], rows, sem).wait()
        pltpu.sync_copy(rows, acc.at[dv], add=True)

        def body(i, carry):
            for u in range(2):
                e = i * 2 + u
                v = rows[e, :]
                db = plsc.load_gather(dv, [lane * 0 + e])
                idxv = db * 8 + lane7
                cur = plsc.load_gather(eacc, [idxv])
                new = jnp.where(msel, jnp.maximum(cur, v),
                                jnp.minimum(cur, v))
                plsc.store_scatter(eacc, [idxv], new, mask=m8)
            return carry

        lax.fori_loop(0, _ECH // 2, body, 0)

    pltpu.sync_copy(cn_h.at[pl.ds(wid * 320, 320)], div)
    pltpu.sync_copy(c13_h, dcb)
    pltpu.sync_copy(dcb, acc.at[div], add=True)
    pltpu.sync_copy(nn_h.at[pl.ds(wid * 320, 320)], div)
    pltpu.sync_copy(c15_h, dcb)
    pltpu.sync_copy(dcb, acc.at[div], add=True)
    pltpu.sync_copy(cd_h.at[pl.ds(wid * 256, 256)], div2)
    pltpu.sync_copy(c14_h, dcb2)
    pltpu.sync_copy(dcb2, acc.at[div2], add=True)

    pltpu.sync_copy(eacc, ex_h.at[pl.ds(wid * _NP8, _NP8)])

    plsc.subcore_barrier()
    pltpu.sync_copy(acc.at[pl.ds(s * _N16S, _N16S)], zb)
    pltpu.sync_copy(zb, sums_h.at[c, pl.ds(s * _N16S, _N16S)])


def _sc_gather_segsum(table, sidx, didx, onesv, zd, zc, nseg, d,
                      with_counts=True):
    eq = sidx.shape[0]
    per = eq // _NW
    nseg16 = nseg // 16

    def body(t_h, s_h, di_h, on_h, zd_h, zc_h, *rest):
        if with_counts:
            (out_h, cnt_h, sv, dv, rows, ov, zbd, zbc, acc, accc, sem) = rest
        else:
            (out_h, sv, dv, rows, ov, zbd, zbc, acc, sem) = rest
        c = lax.axis_index("c")
        s = lax.axis_index("s")
        wid = s * 2 + c
        base = wid * per
        pltpu.sync_copy(zd_h, zbd)
        pltpu.sync_copy(zbd, acc.at[pl.ds(s * nseg16, nseg16)])
        if with_counts:
            pltpu.sync_copy(zc_h, zbc)
            pltpu.sync_copy(zbc, accc.at[pl.ds(s * nseg16, nseg16)])
            pltpu.sync_copy(on_h, ov)
        plsc.subcore_barrier()
        pltpu.sync_copy(s_h.at[pl.ds(base, per)], sv)
        pltpu.sync_copy(di_h.at[pl.ds(base, per)], dv)
        pltpu.async_copy(t_h.at[sv], rows, sem).wait()
        pltpu.sync_copy(rows, acc.at[dv], add=True)
        if with_counts:
            pltpu.sync_copy(ov, accc.at[dv], add=True)
        plsc.subcore_barrier()
        pltpu.sync_copy(acc.at[pl.ds(s * nseg16, nseg16)], zbd)
        pltpu.sync_copy(zbd, out_h.at[c, pl.ds(s * nseg16, nseg16)])
        if with_counts:
            pltpu.sync_copy(accc.at[pl.ds(s * nseg16, nseg16)], zbc)
            pltpu.sync_copy(zbc, cnt_h.at[c, pl.ds(s * nseg16, nseg16)])

    out_type = [_f32((2, nseg, d))]
    scratch = [
        pltpu.VMEM((per,), jnp.int32),
        pltpu.VMEM((per,), jnp.int32),
        pltpu.VMEM((per, d), jnp.float32),
        pltpu.VMEM((per, 8), jnp.float32),
        pltpu.VMEM((nseg16, d), jnp.float32),
        pltpu.VMEM((nseg16, 8), jnp.float32),
        pltpu.VMEM_SHARED((nseg, d), jnp.float32),
    ]
    if with_counts:
        out_type.append(_f32((2, nseg, 8)))
        scratch.append(pltpu.VMEM_SHARED((nseg, 8), jnp.float32))
    scratch.append(pltpu.SemaphoreType.DMA)

    kern = pl.kernel(
        body,
        out_type=tuple(out_type),
        mesh=_mesh,
        scratch_types=scratch,
        compiler_params=_sc_params,
    )
    res = kern(table, sidx, didx, onesv, zd, zc)
    if with_counts:
        return res
    return res[0], None


def _y16_body(x_ref, wbm_ref, s1_ref, s2_ref, cb_ref, out_ref):
    y = jnp.dot(x_ref[...], wbm_ref[...], preferred_element_type=jnp.float32)
    out_ref[...] = (jnp.dot(y, s1_ref[...], preferred_element_type=jnp.float32)
                    + jnp.dot(y * y, s2_ref[...],
                              preferred_element_type=jnp.float32)
                    + cb_ref[...])


def _prologue_body(ng_ref, cg_ref, nW_ref, nb_ref, cW_ref, cb_ref, bt_ref,
                   we1s_ref, be1s_ref, we2s_ref, be2s_ref, wc2s_ref, bc1s_ref,
                   bc2s_ref, x0_ref, c0_ref, wbm_ref, wce_ref):
    x0 = jax.nn.relu(
        jnp.dot(jax.nn.relu(ng_ref[...]), nW_ref[...],
                preferred_element_type=jnp.float32) + nb_ref[...])
    x0_ref[...] = x0
    c0_ref[...] = jax.nn.relu(
        jnp.dot(jax.nn.relu(cg_ref[...]), cW_ref[...],
                preferred_element_type=jnp.float32) + cb_ref[...])
    for l in range(_L):
        hb = jax.nn.relu(jnp.dot(bt_ref[...], we1s_ref[l],
                                 preferred_element_type=jnp.float32) + be1s_ref[l])
        wb = jnp.dot(hb, we2s_ref[l],
                     preferred_element_type=jnp.float32) + be2s_ref[l]
        wbm_ref[l, ...] = wb.reshape(_MAXB, _H, _M).transpose(1, 0, 2).reshape(_H, _MAXB * _M)
        chh = jax.nn.relu(bc1s_ref[l])
        wce_ref[l, ...] = (jnp.dot(chh, wc2s_ref[l],
                                   preferred_element_type=jnp.float32)
                           + bc2s_ref[l]).reshape(_CH, _CH)


def _exfold_body(ex_ref, out_ref):
    exv = ex_ref[...]
    nb8 = exv.shape[1]
    ismax = ((lax.broadcasted_iota(jnp.int32, (1, nb8), 1) >> 2) & 1) == 0
    mxf = jnp.max(jnp.where(ismax, exv, -3.4e38), axis=0, keepdims=True)
    mnf = jnp.min(jnp.where(ismax, 3.4e38, exv), axis=0, keepdims=True)
    out_ref[...] = jnp.where(ismax, mxf, mnf)[None]


def _pna_post_body(x_ref, sums_ref, ex_ref, wp_ref, bp_ref, out_ref):
    ssum = sums_ref[0, :, 0:4] + sums_ref[1, :, 0:4]
    ssq = sums_ref[0, :, 8:12] + sums_ref[1, :, 8:12]
    deg = (sums_ref[0, :, 12] + sums_ref[1, :, 12])[:, None]
    mxp = ex_ref[:, 0:4]
    mnp = ex_ref[:, 4:8]
    degc = jnp.maximum(deg, 1.0)
    mean = ssum / degc
    has = deg > 0.0
    mx = jnp.where(has, mxp, 0.0)
    mn = jnp.where(has, mnp, 0.0)
    std = jnp.sqrt(jnp.clip(ssq / degc - mean * mean, 0.0, None))
    agg = jnp.concatenate([mean, mx, mn, std], axis=-1)
    logd = jnp.log(deg + 1.0)
    amp = logd / _AVG_LOG
    att = jnp.where(logd > 0, _AVG_LOG / jnp.clip(logd, 1e-6, None), 0.0)
    scaled = jnp.concatenate([agg, agg * amp, agg * att], axis=-1)
    h = jnp.concatenate([x_ref[...], scaled], axis=-1)
    out_ref[...] = jax.nn.relu(
        jnp.dot(h, wp_ref[...], preferred_element_type=jnp.float32) + bp_ref[...])


def _c_update_body(c_ref, caggs_ref, sums_ref, wn2c_ref, bn2c_ref, wce_ref,
                   c1_ref, cy_ref):
    cdeg = jnp.maximum(sums_ref[0, :_NC, 13] + sums_ref[1, :_NC, 13],
                       1.0)[:, None]
    cagg = (caggs_ref[0, :_NC, :] + caggs_ref[1, :_NC, :]) / cdeg
    c1 = c_ref[...] + jax.nn.relu(
        jnp.dot(cagg, wn2c_ref[...], preferred_element_type=jnp.float32) + bn2c_ref[...])
    c1_ref[...] = c1
    cy_ref[...] = jnp.dot(c1, wce_ref[...], preferred_element_type=jnp.float32)


def _c_root_body(c_ref, cas_ref, sums_ref, wroot_ref, broot_ref, out_ref):
    ccdeg = jnp.maximum(sums_ref[0, :_NC, 14] + sums_ref[1, :_NC, 14],
                        1.0)[:, None]
    out_ref[...] = (jnp.dot(c_ref[...], wroot_ref[...],
                            preferred_element_type=jnp.float32)
                    + (cas_ref[0, :_NC, :] + cas_ref[1, :_NC, :]) / ccdeg
                    + broot_ref[...])


def _x_update_body(x_ref, naggs_ref, sums_ref, wc2n_ref, bc2n_ref, out_ref):
    ndeg = jnp.maximum(sums_ref[0, :_N, 15] + sums_ref[1, :_N, 15],
                       1.0)[:, None]
    nagg = (naggs_ref[0, :_N, :] + naggs_ref[1, :_N, :]) / ndeg
    out_ref[...] = x_ref[...] + jax.nn.relu(
        jnp.dot(nagg, wc2n_ref[...], preferred_element_type=jnp.float32)
        + bc2n_ref[...])


def _tc(fn, out_shape, *args):
    return pl.pallas_call(fn, out_shape=out_shape)(*args)


def _pad_i32(a, n, fill):
    return jnp.concatenate([a, jnp.full((n - a.shape[0],), fill, jnp.int32)])


def kernel(z, edge_index, bond_type, x_clique, node2clique_index, clique_edge_index, params):
    emb = params["emb"]
    layers = params["layers"]

    src, dst = edge_index[0], edge_index[1]
    csrc, cdst = clique_edge_index[0], clique_edge_index[1]
    nn2c, cn2c = node2clique_index[0], node2clique_index[1]

    ng = jnp.take(emb["node_table"], z, axis=0)
    cg = jnp.take(emb["clique_table"], x_clique, axis=0)

    gidx = (src * _MAXB + bond_type).astype(jnp.int32)
    dst = dst.astype(jnp.int32)

    en2c_p = 10240
    ecc_p = 8192
    nn2c_g = _pad_i32(nn2c.astype(jnp.int32), en2c_p, 0)
    cn2c_s = _pad_i32(cn2c.astype(jnp.int32), en2c_p, _NC)
    cn2c_g = _pad_i32(cn2c.astype(jnp.int32), en2c_p, 0)
    nn2c_s = _pad_i32(nn2c.astype(jnp.int32), en2c_p, _N)
    csrc_g = _pad_i32(csrc.astype(jnp.int32), ecc_p, 0)
    cdst_s = _pad_i32(cdst.astype(jnp.int32), ecc_p, _NC)

    einit = jnp.tile(jnp.array([-3.4e38] * 4 + [3.4e38] * 4, jnp.float32),
                     _NP8 // 8)
    z16 = jnp.zeros((_N16S, 16), jnp.float32)
    c13 = jnp.zeros((320, 16), jnp.float32).at[:, 13].set(1.0)
    c14 = jnp.zeros((256, 16), jnp.float32).at[:, 14].set(1.0)
    c15 = jnp.zeros((320, 16), jnp.float32).at[:, 15].set(1.0)
    s1 = jnp.zeros((32, 128), jnp.float32)
    s2 = jnp.zeros((32, 128), jnp.float32)
    for b in range(_MAXB):
        for q in range(4):
            s1 = s1.at[4 * b + q, 16 * b + q].set(1.0)
            s1 = s1.at[4 * b + q, 16 * b + 4 + q].set(1.0)
            s2 = s2.at[4 * b + q, 16 * b + 8 + q].set(1.0)
    cbrow = jnp.zeros((1, 128), jnp.float32)
    for b in range(_MAXB):
        cbrow = cbrow.at[0, 16 * b + 12].set(1.0)
    ones_n2c = jnp.ones((en2c_p // _NW, 8), jnp.float32)
    ones_ecc = jnp.ones((ecc_p // _NW, 8), jnp.float32)
    zn64 = jnp.zeros((_NSEGC // 16, _H), jnp.float32)
    zc32 = jnp.zeros((_NSEGC // 16, _CH), jnp.float32)
    zn32 = jnp.zeros((_NSEGN // 16, _CH), jnp.float32)
    zc8 = jnp.zeros((_NSEGC // 16, 8), jnp.float32)
    zn8 = jnp.zeros((_NSEGN // 16, 8), jnp.float32)

    we1s = jnp.stack([p["We1"] for p in layers])
    be1s = jnp.stack([p["be1"][None, :] for p in layers])
    we2s = jnp.stack([p["We2"] for p in layers])
    be2s = jnp.stack([p["be2"][None, :] for p in layers])
    wc2s = jnp.stack([p["Wc2"] for p in layers])
    bc1s = jnp.stack([p["bc1"][None, :] for p in layers])
    bc2s = jnp.stack([p["bc2"][None, :] for p in layers])

    x, c, wbm, wce = _tc(
        _prologue_body,
        (_f32((_N, _H)), _f32((_NC, _CH)),
         _f32((_L, _H, _MAXB * _M)), _f32((_L, _CH, _CH))),
        ng, cg, emb["node_linW"], emb["node_linb"][None, :],
        emb["clique_linW"], emb["clique_linb"][None, :], emb["bond_table"],
        we1s, be1s, we2s, be2s, wc2s, bc1s, bc2s)

    for l in range(_L):
        p = layers[l]
        nby = 2000
        y = pl.pallas_call(
            _y16_body,
            grid=(_N // nby,),
            in_specs=[pl.BlockSpec((nby, _H), lambda i: (i, 0)),
                      pl.BlockSpec((_H, _MAXB * _M), lambda i: (0, 0)),
                      pl.BlockSpec((_MAXB * _M, 128), lambda i: (0, 0)),
                      pl.BlockSpec((_MAXB * _M, 128), lambda i: (0, 0)),
                      pl.BlockSpec((1, 128), lambda i: (0, 0))],
            out_specs=pl.BlockSpec((nby, 128), lambda i: (i, 0)),
            out_shape=_f32((_N, 128)),
        )(x, wbm[l], s1, s2, cbrow)
        y16 = y.reshape(_N * _MAXB, 16)
        sums, ex = _sc_edge(y16, gidx, dst, cn2c_s, cdst_s, nn2c_s,
                            c13, c14, c15, einit, z16)
        ex = ex.reshape(_NW, _NP8)
        nb = 2000
        nfold = _N // nb
        exf = pl.pallas_call(
            _exfold_body,
            grid=(nfold,),
            in_specs=[pl.BlockSpec((_NW, nb * 8), lambda i: (0, i))],
            out_specs=pl.BlockSpec((1, 1, nb * 8), lambda i: (i, 0, 0)),
            out_shape=_f32((nfold, 1, nb * 8)),
        )(ex).reshape(_N, 8)
        row = lambda w: pl.BlockSpec((nb, w), lambda i: (i, 0))
        x = pl.pallas_call(
            _pna_post_body,
            grid=(_N // nb,),
            in_specs=[row(_H),
                      pl.BlockSpec((2, nb, 16), lambda i: (0, i, 0)),
                      row(8),
                      pl.BlockSpec((_H + 12 * _M, _H), lambda i: (0, 0)),
                      pl.BlockSpec((1, _H), lambda i: (0, 0))],
            out_specs=row(_H),
            out_shape=_f32((_N, _H)),
        )(x, sums, exf, p["Wp"], p["bp"][None, :])

        caggs, _ = _sc_gather_segsum(x, nn2c_g, cn2c_s, ones_n2c,
                                     zn64, zc8, _NSEGC, _H,
                                     with_counts=False)
        c, cy = _tc(_c_update_body, (_f32((_NC, _CH)), _f32((_NC, _CH))),
                    c, caggs, sums,
                    p["Wn2c"], p["bn2c"][None, :], wce[l])

        cas, _ = _sc_gather_segsum(cy, csrc_g, cdst_s, ones_ecc,
                                   zc32, zc8, _NSEGC, _CH,
                                   with_counts=False)
        c = _tc(_c_root_body, _f32((_NC, _CH)),
                c, cas, sums,
                p["Wroot"], p["broot"][None, :])

        naggs, _ = _sc_gather_segsum(c, cn2c_g, nn2c_s, ones_n2c,
                                     zn32, zn8, _NSEGN, _CH,
                                     with_counts=False)
        x = _tc(_x_update_body, _f32((_N, _H)),
                x, naggs, sums, p["Wc2n"], p["bc2n"][None, :])

    return x, c
